# baseline (speedup 1.0000x reference)
import jax
import jax.numpy as jnp
from jax import lax
from jax.experimental import pallas as pl
from jax.experimental.pallas import tpu as pltpu
from jax.experimental.pallas import tpu_sc as plsc

B = 16
N = 4096
R = 128
BN = B * N
NC = 2
NS = 16
NW = NC * NS
ROWS_PER_W = BN // NW
CHUNK = 128
NCHUNK = ROWS_PER_W // CHUNK
L = 16
G = R // L


def _sc_body(idx_hbm, msk_hbm, pri_hbm, tab_hbm, cond_hbm, out_hbm,
             idx_v, msk_v, g_v, a_v, p_v, sem):
    wid = lax.axis_index("s") * NC + lax.axis_index("c")
    w_base = wid * ROWS_PER_W

    for c in range(NCHUNK):
        base = w_base + c * CHUNK
        pltpu.sync_copy(idx_hbm.at[pl.ds(base, CHUNK)], idx_v)
        pltpu.sync_copy(msk_hbm.at[pl.ds(base, CHUNK)], msk_v)
        cp_g = pltpu.async_copy(cond_hbm.at[idx_v], g_v, sem)
        cp_a = pltpu.async_copy(tab_hbm.at[msk_v], a_v, sem)
        pltpu.sync_copy(pri_hbm.at[pl.ds(base, CHUNK), :], p_v)
        cp_g.wait()
        cp_a.wait()

        def row(r, _):
            for j in range(G):
                sl = pl.ds(j * L, L)
                g_v[r, sl] = g_v[r, sl] + a_v[r, sl] + p_v[r, sl]
            return 0

        lax.fori_loop(0, CHUNK, row, 0)
        pltpu.sync_copy(g_v, out_hbm.at[pl.ds(base, CHUNK), :])


@jax.jit
def _sc_call(idx, msk, pri2d, tab, cond):
    mesh = plsc.VectorSubcoreMesh(core_axis_name="c", subcore_axis_name="s")
    return pl.kernel(
        _sc_body,
        out_type=jax.ShapeDtypeStruct((BN, R), jnp.float32),
        mesh=mesh,
        scratch_types=[
            pltpu.VMEM((CHUNK,), jnp.int32),
            pltpu.VMEM((CHUNK,), jnp.int32),
            pltpu.VMEM((CHUNK, R), jnp.float32),
            pltpu.VMEM((CHUNK, R), jnp.float32),
            pltpu.VMEM((CHUNK, R), jnp.float32),
            pltpu.SemaphoreType.DMA,
        ],
    )(idx, msk, pri2d, tab, cond)


def kernel(cond_inds, node_mask, full_logit_priors, unconditionals, conditionals):
    idx = jnp.where(node_mask, cond_inds, 0).reshape(BN)
    msk = node_mask.reshape(BN).astype(jnp.int32)
    pri2d = full_logit_priors.reshape(BN, R)
    t = -conditionals[0] - jnp.float32(100000.0)
    tab = jnp.stack([t, unconditionals])
    out2d = _sc_call(idx, msk, pri2d, tab, conditionals)
    return out2d.reshape(B, N * R), full_logit_priors.reshape(B, N * R)


# speedup vs baseline: 8.7711x; 8.7711x over previous
import jax
import jax.numpy as jnp
from jax import lax
from jax.experimental import pallas as pl
from jax.experimental.pallas import tpu as pltpu
from jax.experimental.pallas import tpu_sc as plsc

B = 16
N = 4096
R = 128
BN = B * N
NC = 2
NS = 16
NW = NC * NS
ROWS_PER_W = BN // NW
CHUNK = 128
NCHUNK = ROWS_PER_W // CHUNK
L = 16
G = R // L

_SPLAT_DNUMS = lax.GatherDimensionNumbers(
    offset_dims=(), collapsed_slice_dims=(0,), start_index_map=(0,))


def _splat(vec, lane):
    idxv = jnp.full((L,), 0, jnp.int32) + lane
    return lax.gather(vec, idxv[:, None], _SPLAT_DNUMS, (1,),
                      mode=lax.GatherScatterMode.PROMISE_IN_BOUNDS)


def _sc_body(idx_hbm, msk_hbm, pri_hbm, u_hbm, cond_hbm, out_hbm,
             idx_v, msk_v, u_v, g_v, p_v, o_v,
             sem_g, sem_p, sem_o):
    wid = lax.axis_index("s") * NC + lax.axis_index("c")
    w_base = wid * ROWS_PER_W

    pltpu.sync_copy(idx_hbm.at[pl.ds(w_base, ROWS_PER_W)], idx_v)
    pltpu.sync_copy(msk_hbm.at[pl.ds(w_base, ROWS_PER_W)], msk_v)
    pltpu.sync_copy(u_hbm, u_v)
    u_regs = [u_v[pl.ds(j * L, L)] + 100000.0 for j in range(G)]

    def start_in(c):
        buf = c % 2
        dg = pltpu.async_copy(
            cond_hbm.at[idx_v.at[pl.ds(c * CHUNK, CHUNK)]], g_v.at[buf],
            sem_g.at[buf])
        dp = pltpu.async_copy(
            pri_hbm.at[pl.ds(w_base + c * CHUNK, CHUNK), :], p_v.at[buf],
            sem_p.at[buf])
        return dg, dp

    in_flight = {0: start_in(0), 1: start_in(1)}
    out_flight = {}

    for c in range(NCHUNK):
        buf = c % 2
        dg, dp = in_flight.pop(c)
        dg.wait()
        dp.wait()
        if c - 2 in out_flight:
            out_flight.pop(c - 2).wait()

        def row(r, _):
            grp = msk_v[pl.ds(c * CHUNK + (r & ~15), L)]
            m = _splat(grp, r & 15)
            for j in range(G):
                sl = pl.ds(j * L, L)
                o_v[buf, r, sl] = (
                    m * (g_v[buf, r, sl] + u_regs[j]) - 100000.0
                    + p_v[buf, r, sl])
            return 0

        lax.fori_loop(0, CHUNK, row, 0)

        out_flight[c] = pltpu.async_copy(
            o_v.at[buf], out_hbm.at[pl.ds(w_base + c * CHUNK, CHUNK), :],
            sem_o.at[buf])
        if c + 2 < NCHUNK:
            in_flight[c + 2] = start_in(c + 2)

    for d in out_flight.values():
        d.wait()


@jax.jit
def _sc_call(idx, msk, pri2d, u, cond):
    mesh = plsc.VectorSubcoreMesh(core_axis_name="c", subcore_axis_name="s")
    return pl.kernel(
        _sc_body,
        out_type=jax.ShapeDtypeStruct((BN, R), jnp.float32),
        mesh=mesh,
        scratch_types=[
            pltpu.VMEM((ROWS_PER_W,), jnp.int32),
            pltpu.VMEM((ROWS_PER_W,), jnp.float32),
            pltpu.VMEM((R,), jnp.float32),
            pltpu.VMEM((2, CHUNK, R), jnp.float32),
            pltpu.VMEM((2, CHUNK, R), jnp.float32),
            pltpu.VMEM((2, CHUNK, R), jnp.float32),
            pltpu.SemaphoreType.DMA((2,)),
            pltpu.SemaphoreType.DMA((2,)),
            pltpu.SemaphoreType.DMA((2,)),
        ],
    )(idx, msk, pri2d, u, cond)


def kernel(cond_inds, node_mask, full_logit_priors, unconditionals, conditionals):
    idx = cond_inds.reshape(BN)
    msk = node_mask.reshape(BN).astype(jnp.float32)
    pri2d = full_logit_priors.reshape(BN, R)
    out2d = _sc_call(idx, msk, pri2d, unconditionals, conditionals)
    return out2d.reshape(B, N * R), full_logit_priors.reshape(B, N * R)
